# Initial kernel scaffold; baseline (speedup 1.0000x reference)
import jax
import jax.numpy as jnp
from jax.experimental import pallas as pl


def kernel(token_ids, embed_table, W, b):
    raise NotImplementedError("write your pallas kernel here")


# speedup vs baseline: 3.7821x; 3.7821x over previous
import functools

import jax
import jax.numpy as jnp
from jax import lax
from jax.experimental import pallas as pl
from jax.experimental.pallas import tpu as pltpu
from jax.experimental.pallas import tpu_sc as plsc

_LANES = 128
_K = 8


def _proj_body(e_ref, w_ref, b_ref, o_ref):
    o_ref[...] = (
        jnp.dot(e_ref[...], w_ref[...], preferred_element_type=jnp.float32)
        + b_ref[...]
    )


def _project_table(embed_table, W, b):
    V, E = embed_table.shape
    D = W.shape[1]
    blk = 2000
    assert V % blk == 0
    return pl.pallas_call(
        _proj_body,
        grid=(V // blk,),
        in_specs=[
            pl.BlockSpec((blk, E), lambda i: (i, 0)),
            pl.BlockSpec((E, D), lambda i: (0, 0)),
            pl.BlockSpec((1, D), lambda i: (0, 0)),
        ],
        out_specs=pl.BlockSpec((blk, D), lambda i: (i, 0)),
        out_shape=jax.ShapeDtypeStruct((V, D), jnp.float32),
    )(embed_table, W, b.reshape(1, D))


@functools.lru_cache(maxsize=None)
def _make_gather(V, D, B):
    info = plsc.get_sparse_core_info()
    NC, NS = info.num_cores, info.num_subcores
    NW = NC * NS
    assert B % (NW * _LANES * _K) == 0
    b_per_w = B // NW
    n_chunks = b_per_w // _LANES
    groups = n_chunks // _K
    mesh = plsc.VectorSubcoreMesh(core_axis_name="c", subcore_axis_name="s")

    @functools.partial(
        pl.kernel,
        mesh=mesh,
        out_type=jax.ShapeDtypeStruct((B, D), jnp.float32),
        scratch_types=[
            pltpu.VMEM((n_chunks, _LANES), jnp.int32),
            pltpu.VMEM((_K * _LANES, D), jnp.float32),
            pltpu.SemaphoreType.DMA,
        ],
        compiler_params=pltpu.CompilerParams(use_tc_tiling_on_sc=False),
    )
    def gather_kernel(idx_hbm, table_hbm, out_hbm, idx_v, rows_v, sem):
        wid = lax.axis_index("s") * NC + lax.axis_index("c")
        row0 = wid * n_chunks
        tok0 = wid * b_per_w
        pltpu.sync_copy(idx_hbm.at[pl.ds(row0, n_chunks)], idx_v)

        def group(g, carry):
            cps = []
            for j in range(_K):
                cps.append(
                    pltpu.async_copy(
                        table_hbm.at[idx_v.at[g * _K + j]],
                        rows_v.at[pl.ds(j * _LANES, _LANES)],
                        sem,
                    )
                )
            for cp in cps:
                cp.wait()
            pltpu.sync_copy(
                rows_v,
                out_hbm.at[pl.ds(tok0 + g * (_K * _LANES), _K * _LANES)],
            )
            return carry

        lax.fori_loop(0, groups, group, 0)

    return gather_kernel


def kernel(token_ids, embed_table, W, b):
    Bt, S = token_ids.shape
    V, E = embed_table.shape
    D = W.shape[1]
    proj = _project_table(embed_table, W, b)
    B = Bt * S
    ids = token_ids.reshape(B // _LANES, _LANES).astype(jnp.int32)
    out = _make_gather(V, D, B)(ids, proj)
    return out.reshape(Bt, S, D)


# speedup vs baseline: 3.8399x; 1.0153x over previous
import functools

import jax
import jax.numpy as jnp
from jax import lax
from jax.experimental import pallas as pl
from jax.experimental.pallas import tpu as pltpu
from jax.experimental.pallas import tpu_sc as plsc

_LANES = 128
_K = 4


def _proj_body(e_ref, w_ref, b_ref, o_ref):
    o_ref[...] = (
        jnp.dot(e_ref[...], w_ref[...], preferred_element_type=jnp.float32)
        + b_ref[...]
    )


def _project_table(embed_table, W, b):
    V, E = embed_table.shape
    D = W.shape[1]
    blk = 2000
    assert V % blk == 0
    return pl.pallas_call(
        _proj_body,
        grid=(V // blk,),
        in_specs=[
            pl.BlockSpec((blk, E), lambda i: (i, 0)),
            pl.BlockSpec((E, D), lambda i: (0, 0)),
            pl.BlockSpec((1, D), lambda i: (0, 0)),
        ],
        out_specs=pl.BlockSpec((blk, D), lambda i: (i, 0)),
        out_shape=jax.ShapeDtypeStruct((V, D), jnp.float32),
    )(embed_table, W, b.reshape(1, D))


@functools.lru_cache(maxsize=None)
def _make_gather(V, D, B):
    info = plsc.get_sparse_core_info()
    NC, NS = info.num_cores, info.num_subcores
    NW = NC * NS
    assert B % (NW * _LANES * _K) == 0
    b_per_w = B // NW
    n_chunks = b_per_w // _LANES
    groups = n_chunks // _K
    mesh = plsc.VectorSubcoreMesh(core_axis_name="c", subcore_axis_name="s")

    @functools.partial(
        pl.kernel,
        mesh=mesh,
        out_type=jax.ShapeDtypeStruct((B, D), jnp.float32),
        scratch_types=[
            pltpu.VMEM((n_chunks, _LANES), jnp.int32),
            pltpu.VMEM((2, _K * _LANES, D), jnp.float32),
            pltpu.SemaphoreType.DMA,
            pltpu.SemaphoreType.DMA,
        ],
        compiler_params=pltpu.CompilerParams(use_tc_tiling_on_sc=False),
    )
    def gather_kernel(idx_hbm, table_hbm, out_hbm, idx_v, rows_v, sem_g, sem_o):
        wid = lax.axis_index("s") * NC + lax.axis_index("c")
        row0 = wid * n_chunks
        tok0 = wid * b_per_w
        grp_rows = _K * _LANES
        pltpu.sync_copy(idx_hbm.at[pl.ds(row0, n_chunks)], idx_v)

        def fire(g, slot):
            for j in range(_K):
                pltpu.async_copy(
                    table_hbm.at[idx_v.at[g * _K + j]],
                    rows_v.at[slot, pl.ds(j * _LANES, _LANES)],
                    sem_g,
                )

        def drain_gather(slot):
            pltpu.make_async_copy(
                table_hbm.at[pl.ds(0, grp_rows)], rows_v.at[slot], sem_g
            ).wait()

        def drain_out():
            pltpu.make_async_copy(
                rows_v.at[0], out_hbm.at[pl.ds(0, grp_rows)], sem_o
            ).wait()

        fire(0, 0)

        def body(g, carry):
            slot = lax.rem(g, 2)
            nslot = 1 - slot

            @pl.when(g + 1 < groups)
            def _():
                @pl.when(g >= 1)
                def _():
                    drain_out()

                fire(g + 1, nslot)

            drain_gather(slot)
            pltpu.async_copy(
                rows_v.at[slot],
                out_hbm.at[pl.ds(tok0 + g * grp_rows, grp_rows)],
                sem_o,
            )
            return carry

        lax.fori_loop(0, groups, body, 0)
        drain_out()
        drain_out()

    return gather_kernel


def kernel(token_ids, embed_table, W, b):
    Bt, S = token_ids.shape
    V, E = embed_table.shape
    D = W.shape[1]
    proj = _project_table(embed_table, W, b)
    B = Bt * S
    ids = token_ids.reshape(B // _LANES, _LANES).astype(jnp.int32)
    out = _make_gather(V, D, B)(ids, proj)
    return out.reshape(Bt, S, D)


# speedup vs baseline: 4.4252x; 1.1524x over previous
import functools

import jax
import jax.numpy as jnp
from jax import lax
from jax.experimental import pallas as pl
from jax.experimental.pallas import tpu as pltpu
from jax.experimental.pallas import tpu_sc as plsc

_LANES = 128
_K = 4


def _proj_body(et_ref, eb_ref, w_ref, b_ref, o_ref):
    top = jnp.dot(et_ref[...], w_ref[...], preferred_element_type=jnp.float32)
    bot = jnp.dot(eb_ref[...], w_ref[...], preferred_element_type=jnp.float32)
    o_ref[...] = jnp.concatenate([top, bot], axis=1) + jnp.concatenate(
        [b_ref[...], b_ref[...]], axis=1
    )


def _project_table_packed(embed_table, W, b):
    V, E = embed_table.shape
    D = W.shape[1]
    H = V // 2
    blk = 2000
    return pl.pallas_call(
        _proj_body,
        grid=(H // blk,),
        in_specs=[
            pl.BlockSpec((blk, E), lambda i: (i, 0)),
            pl.BlockSpec((blk, E), lambda i, _h=H // blk: (i + _h, 0)),
            pl.BlockSpec((E, D), lambda i: (0, 0)),
            pl.BlockSpec((1, D), lambda i: (0, 0)),
        ],
        out_specs=pl.BlockSpec((blk, 2 * D), lambda i: (i, 0)),
        out_shape=jax.ShapeDtypeStruct((H, 2 * D), jnp.float32),
    )(embed_table, embed_table, W, b.reshape(1, D))


@functools.lru_cache(maxsize=None)
def _make_gather(V, D, B):
    info = plsc.get_sparse_core_info()
    NC, NS = info.num_cores, info.num_subcores
    NW = NC * NS
    b_per_w = B // NW
    n_chunks = b_per_w // _LANES
    groups = n_chunks // _K
    mesh = plsc.VectorSubcoreMesh(core_axis_name="c", subcore_axis_name="s")

    @functools.partial(
        pl.kernel,
        mesh=mesh,
        out_type=jax.ShapeDtypeStruct((B, D), jnp.float32),
        scratch_types=[
            pltpu.VMEM((n_chunks, _LANES), jnp.int32),
            pltpu.VMEM((2, _K * _LANES, D), jnp.float32),
            pltpu.SemaphoreType.DMA,
            pltpu.SemaphoreType.DMA,
        ],
        compiler_params=pltpu.CompilerParams(use_tc_tiling_on_sc=False),
    )
    def gather_kernel(idx_hbm, table_hbm, out_hbm, idx_v, rows_v, sem_g, sem_o):
        wid = lax.axis_index("s") * NC + lax.axis_index("c")
        row0 = wid * n_chunks
        tok0 = wid * b_per_w
        grp_rows = _K * _LANES
        pltpu.sync_copy(idx_hbm.at[pl.ds(row0, n_chunks)], idx_v)

        def fire(g, slot):
            for j in range(_K):
                pltpu.async_copy(
                    table_hbm.at[idx_v.at[g * _K + j]],
                    rows_v.at[slot, pl.ds(j * _LANES, _LANES)],
                    sem_g,
                )

        def drain_gather(slot):
            pltpu.make_async_copy(
                table_hbm.at[pl.ds(0, grp_rows)], rows_v.at[slot], sem_g
            ).wait()

        def drain_out():
            pltpu.make_async_copy(
                rows_v.at[0], out_hbm.at[pl.ds(0, grp_rows)], sem_o
            ).wait()

        fire(0, 0)

        def body(g, carry):
            slot = lax.rem(g, 2)
            nslot = 1 - slot

            @pl.when(g + 1 < groups)
            def _():
                @pl.when(g >= 1)
                def _():
                    drain_out()

                fire(g + 1, nslot)

            drain_gather(slot)
            pltpu.async_copy(
                rows_v.at[slot],
                out_hbm.at[pl.ds(tok0 + g * grp_rows, grp_rows)],
                sem_o,
            )
            return carry

        lax.fori_loop(0, groups, body, 0)
        drain_out()
        drain_out()

    return gather_kernel


def _trans_body(x_ref, o_ref):
    t = jnp.swapaxes(x_ref[0], 0, 1)
    o_ref[0] = jnp.concatenate([t[:64], t[64:]], axis=1)


def _transpose_out(x, S, D, Bt):
    return pl.pallas_call(
        _trans_body,
        grid=(S,),
        in_specs=[pl.BlockSpec((1, Bt // 2, 2 * D), lambda s: (s, 0, 0))],
        out_specs=pl.BlockSpec((1, D, Bt), lambda s: (s, 0, 0)),
        out_shape=jax.ShapeDtypeStruct((S, D, Bt), jnp.float32),
    )(x)


def kernel(token_ids, embed_table, W, b):
    Bt, S = token_ids.shape
    V, E = embed_table.shape
    D = W.shape[1]
    H = V // 2
    B = Bt * S

    packed = _project_table_packed(embed_table, W, b)

    ids = token_ids.astype(jnp.int32)
    ids2 = 2 * (ids % H) + ids // H
    half = Bt // 2
    idx = jnp.stack([ids2[:half].T, ids2[half:].T], axis=-1)
    idx = idx.reshape(B // _LANES, _LANES)

    flat = _make_gather(V, D, B)(idx, packed.reshape(V, D))
    x = flat.reshape(S, half, 2 * D)
    outp = _transpose_out(x, S, D, Bt)
    return outp.transpose(2, 0, 1)


# speedup vs baseline: 6.6369x; 1.4998x over previous
import functools

import jax
import jax.numpy as jnp
from jax import lax
from jax.experimental import pallas as pl
from jax.experimental.pallas import tpu as pltpu
from jax.experimental.pallas import tpu_sc as plsc

_LANES = 128
_K = 4


def _proj_body(et_ref, eb_ref, w_ref, b_ref, o_ref):
    top = jnp.dot(et_ref[...], w_ref[...], preferred_element_type=jnp.float32)
    bot = jnp.dot(eb_ref[...], w_ref[...], preferred_element_type=jnp.float32)
    o_ref[...] = jnp.concatenate([top, bot], axis=1) + jnp.concatenate(
        [b_ref[...], b_ref[...]], axis=1
    )


def _project_table_packed(embed_table, W, b):
    V, E = embed_table.shape
    D = W.shape[1]
    H = V // 2
    blk = 2000
    return pl.pallas_call(
        _proj_body,
        grid=(H // blk,),
        in_specs=[
            pl.BlockSpec((blk, E), lambda i: (i, 0)),
            pl.BlockSpec((blk, E), lambda i, _h=H // blk: (i + _h, 0)),
            pl.BlockSpec((E, D), lambda i: (0, 0)),
            pl.BlockSpec((1, D), lambda i: (0, 0)),
        ],
        out_specs=pl.BlockSpec((blk, 2 * D), lambda i: (i, 0)),
        out_shape=jax.ShapeDtypeStruct((H, 2 * D), jnp.float32),
    )(embed_table, embed_table, W, b.reshape(1, D))


@functools.lru_cache(maxsize=None)
def _make_gather(V, D, B):
    info = plsc.get_sparse_core_info()
    NC, NS = info.num_cores, info.num_subcores
    NW = NC * NS
    b_per_w = B // NW
    n_chunks = b_per_w // _LANES
    groups = n_chunks // _K
    mesh = plsc.VectorSubcoreMesh(core_axis_name="c", subcore_axis_name="s")

    @functools.partial(
        pl.kernel,
        mesh=mesh,
        out_type=jax.ShapeDtypeStruct((B, D), jnp.float32),
        scratch_types=[
            pltpu.VMEM((n_chunks, _LANES), jnp.int32),
            pltpu.VMEM((2, _K * _LANES, D), jnp.float32),
            pltpu.SemaphoreType.DMA,
            pltpu.SemaphoreType.DMA,
        ],
        compiler_params=pltpu.CompilerParams(use_tc_tiling_on_sc=False),
    )
    def gather_kernel(idx_hbm, table_hbm, out_hbm, idx_v, rows_v, sem_g, sem_o):
        wid = lax.axis_index("s") * NC + lax.axis_index("c")
        row0 = wid * n_chunks
        tok0 = wid * b_per_w
        grp_rows = _K * _LANES
        pltpu.sync_copy(idx_hbm.at[pl.ds(row0, n_chunks)], idx_v)

        def fire(g, slot):
            for j in range(_K):
                pltpu.async_copy(
                    table_hbm.at[idx_v.at[g * _K + j]],
                    rows_v.at[slot, pl.ds(j * _LANES, _LANES)],
                    sem_g,
                )

        def drain_gather(slot):
            pltpu.make_async_copy(
                table_hbm.at[pl.ds(0, grp_rows)], rows_v.at[slot], sem_g
            ).wait()

        def drain_out():
            pltpu.make_async_copy(
                rows_v.at[0], out_hbm.at[pl.ds(0, grp_rows)], sem_o
            ).wait()

        fire(0, 0)

        def body(g, carry):
            slot = lax.rem(g, 2)
            nslot = 1 - slot

            @pl.when(g + 1 < groups)
            def _():
                @pl.when(g >= 1)
                def _():
                    drain_out()

                fire(g + 1, nslot)

            drain_gather(slot)
            pltpu.async_copy(
                rows_v.at[slot],
                out_hbm.at[pl.ds(tok0 + g * grp_rows, grp_rows)],
                sem_o,
            )
            return carry

        lax.fori_loop(0, groups, body, 0)
        drain_out()
        drain_out()

    return gather_kernel


def _trans_body(x_ref, o_ref):
    t = jnp.swapaxes(x_ref[0], 0, 1)
    D = t.shape[0] // 2
    o_ref[0] = jnp.concatenate([t[:D], t[D:]], axis=1)


def _transpose_out(x, S, D, Bt):
    return pl.pallas_call(
        _trans_body,
        grid=(S,),
        in_specs=[pl.BlockSpec((1, Bt // 2, 2 * D), lambda s: (s, 0, 0))],
        out_specs=pl.BlockSpec((1, D, Bt), lambda s: (s, 0, 0)),
        out_shape=jax.ShapeDtypeStruct((S, D, Bt), jnp.float32),
    )(x)


def kernel(token_ids, embed_table, W, b):
    Bt, S = token_ids.shape
    V, E = embed_table.shape
    D = W.shape[1]
    H = V // 2
    B = Bt * S

    packed = _project_table_packed(embed_table, W, b)

    ids = token_ids.astype(jnp.int32)
    half = Bt // 2
    c = jnp.arange(Bt, dtype=jnp.int32)
    rows = (c >> 1) + (c & 1) * half
    ids_p = 2 * (ids[rows, :] % H) + ids[rows, :] // H
    idx = ids_p.T.reshape(B // _LANES, _LANES)

    flat = _make_gather(V, D, B)(idx, packed.reshape(V, D))
    x = flat.reshape(S, half, 2 * D)
    outp = _transpose_out(x, S, D, Bt)
    return outp.transpose(2, 0, 1)


# speedup vs baseline: 7.9091x; 1.1917x over previous
import functools

import jax
import jax.numpy as jnp
from jax import lax
from jax.experimental import pallas as pl
from jax.experimental.pallas import tpu as pltpu
from jax.experimental.pallas import tpu_sc as plsc

_LANES = 128
_K = 4


def _proj_body(et_ref, eb_ref, w_ref, b_ref, o_ref):
    top = jnp.dot(et_ref[...], w_ref[...], preferred_element_type=jnp.float32)
    bot = jnp.dot(eb_ref[...], w_ref[...], preferred_element_type=jnp.float32)
    o_ref[...] = jnp.concatenate([top, bot], axis=1) + jnp.concatenate(
        [b_ref[...], b_ref[...]], axis=1
    )


def _project_table_packed(embed_table, W, b):
    V, E = embed_table.shape
    D = W.shape[1]
    H = V // 2
    blk = 2000
    return pl.pallas_call(
        _proj_body,
        grid=(H // blk,),
        in_specs=[
            pl.BlockSpec((blk, E), lambda i: (i, 0)),
            pl.BlockSpec((blk, E), lambda i, _h=H // blk: (i + _h, 0)),
            pl.BlockSpec((E, D), lambda i: (0, 0)),
            pl.BlockSpec((1, D), lambda i: (0, 0)),
        ],
        out_specs=pl.BlockSpec((blk, 2 * D), lambda i: (i, 0)),
        out_shape=jax.ShapeDtypeStruct((H, 2 * D), jnp.float32),
    )(embed_table, embed_table, W, b.reshape(1, D))


@functools.lru_cache(maxsize=None)
def _make_gather(V, D, B, idx_row0, K):
    info = plsc.get_sparse_core_info()
    NC, NS = info.num_cores, info.num_subcores
    NW = NC * NS
    b_per_w = B // NW
    n_chunks = b_per_w // _LANES
    groups = n_chunks // K
    mesh = plsc.VectorSubcoreMesh(core_axis_name="c", subcore_axis_name="s")

    @functools.partial(
        pl.kernel,
        mesh=mesh,
        out_type=jax.ShapeDtypeStruct((B, D), jnp.float32),
        scratch_types=[
            pltpu.VMEM((n_chunks, _LANES), jnp.int32),
            pltpu.VMEM((2, K * _LANES, D), jnp.float32),
            pltpu.SemaphoreType.DMA,
            pltpu.SemaphoreType.DMA,
        ],
        compiler_params=pltpu.CompilerParams(use_tc_tiling_on_sc=False),
    )
    def gather_kernel(idx_hbm, table_hbm, out_hbm, idx_v, rows_v, sem_g, sem_o):
        wid = lax.axis_index("s") * NC + lax.axis_index("c")
        row0 = idx_row0 + wid * n_chunks
        tok0 = wid * b_per_w
        grp_rows = K * _LANES
        pltpu.sync_copy(idx_hbm.at[pl.ds(row0, n_chunks)], idx_v)

        def fire(g, slot):
            for j in range(K):
                pltpu.async_copy(
                    table_hbm.at[idx_v.at[g * K + j]],
                    rows_v.at[slot, pl.ds(j * _LANES, _LANES)],
                    sem_g,
                )

        def drain_gather(slot):
            pltpu.make_async_copy(
                table_hbm.at[pl.ds(0, grp_rows)], rows_v.at[slot], sem_g
            ).wait()

        def drain_out():
            pltpu.make_async_copy(
                rows_v.at[0], out_hbm.at[pl.ds(0, grp_rows)], sem_o
            ).wait()

        fire(0, 0)

        def body(g, carry):
            slot = lax.rem(g, 2)
            nslot = 1 - slot

            @pl.when(g + 1 < groups)
            def _():
                @pl.when(g >= 1)
                def _():
                    drain_out()

                fire(g + 1, nslot)

            drain_gather(slot)
            pltpu.async_copy(
                rows_v.at[slot],
                out_hbm.at[pl.ds(tok0 + g * grp_rows, grp_rows)],
                sem_o,
            )
            return carry

        lax.fori_loop(0, groups, body, 0)
        drain_out()
        drain_out()

    return gather_kernel


_TROWS = 2


def _trans_body(x_ref, o_ref):
    for r in range(_TROWS):
        t = jnp.swapaxes(x_ref[r], 0, 1)
        D = t.shape[0] // 2
        o_ref[r] = jnp.concatenate([t[:D], t[D:]], axis=1)


def _trans_body_acc(x_ref, carry_ref, o_ref):
    del carry_ref
    _trans_body(x_ref, o_ref)


def _transpose_slab(x, S, D, Bt, s0, carry):
    S_slab = x.shape[0]
    grid = (S_slab // _TROWS,)
    x_spec = pl.BlockSpec((_TROWS, Bt // 2, 2 * D), lambda j: (j, 0, 0))
    o_spec = pl.BlockSpec(
        (_TROWS, D, Bt), lambda j, _b=s0 // _TROWS: (_b + j, 0, 0)
    )
    out_shape = jax.ShapeDtypeStruct((S, D, Bt), jnp.float32)
    if carry is None:
        return pl.pallas_call(
            _trans_body,
            grid=grid,
            in_specs=[x_spec],
            out_specs=o_spec,
            out_shape=out_shape,
        )(x)
    return pl.pallas_call(
        _trans_body_acc,
        grid=grid,
        in_specs=[x_spec, pl.BlockSpec(memory_space=pl.ANY)],
        out_specs=o_spec,
        out_shape=out_shape,
        input_output_aliases={1: 0},
    )(x, carry)


def kernel(token_ids, embed_table, W, b):
    Bt, S = token_ids.shape
    V, E = embed_table.shape
    D = W.shape[1]
    H = V // 2
    B = Bt * S

    packed = _project_table_packed(embed_table, W, b)

    ids = token_ids.astype(jnp.int32)
    half = Bt // 2
    c = jnp.arange(Bt, dtype=jnp.int32)
    rows = (c >> 1) + (c & 1) * half
    ids_p = 2 * (ids[rows, :] % H) + ids[rows, :] // H
    idx = ids_p.T.reshape(B // _LANES, _LANES)

    SLABS = 4
    S_slab = S // SLABS
    rows_per_slab = S_slab * Bt // _LANES
    K = 5 if (rows_per_slab // 32) % 4 else 4
    table = packed.reshape(V, D)
    outp = None
    for k in range(SLABS):
        flat = _make_gather(V, D, S_slab * Bt, k * rows_per_slab, K)(idx, table)
        x = flat.reshape(S_slab, half, 2 * D)
        outp = _transpose_slab(x, S, D, Bt, k * S_slab, outp)
    return outp.transpose(2, 0, 1)
